# Initial kernel scaffold; baseline (speedup 1.0000x reference)
import jax
import jax.numpy as jnp
from jax.experimental import pallas as pl


def kernel(x, idx_swap):
    raise NotImplementedError("write your pallas kernel here")


# speedup vs baseline: 1.0296x; 1.0296x over previous
import jax
import jax.numpy as jnp
from jax.experimental import pallas as pl
from jax.experimental.pallas import tpu as pltpu

_EPS = 1e-05


def _body(idx_ref, xa_ref, xb_ref, out_ref):
    xa = xa_ref[0]
    xb = xb_ref[0]
    c = xa.shape[0]
    mean_a = jnp.mean(xa, axis=0, keepdims=True)
    mean_b = jnp.mean(xb, axis=0, keepdims=True)
    da = xa - mean_a
    db = xb - mean_b
    var_a = jnp.sum(da * da, axis=0, keepdims=True) * (1.0 / (c - 1))
    var_b = jnp.sum(db * db, axis=0, keepdims=True) * (1.0 / (c - 1))
    scale = jnp.sqrt((var_a + _EPS) / (var_b + _EPS))
    out_ref[0] = db * scale + mean_a


def kernel(x, idx_swap):
    n, c, h, w = x.shape
    hw = h * w
    x3 = x.reshape(n, c, hw)
    grid_spec = pltpu.PrefetchScalarGridSpec(
        num_scalar_prefetch=1,
        grid=(n,),
        in_specs=[
            pl.BlockSpec((1, c, hw), lambda i, idx_ref: (i, 0, 0)),
            pl.BlockSpec((1, c, hw), lambda i, idx_ref: (idx_ref[i], 0, 0)),
        ],
        out_specs=pl.BlockSpec((1, c, hw), lambda i, idx_ref: (i, 0, 0)),
    )
    out = pl.pallas_call(
        _body,
        grid_spec=grid_spec,
        out_shape=jax.ShapeDtypeStruct((n, c, hw), x.dtype),
    )(idx_swap, x3, x3)
    return out.reshape(n, c, h, w)
